# Initial kernel scaffold; baseline (speedup 1.0000x reference)
import jax
import jax.numpy as jnp
from jax.experimental import pallas as pl


def kernel(x, W0_0, b0_0, W0_1, b0_1, W0_2, b0_2, W1_0, b1_0, W1_1, b1_1, W1_2, b1_2, Wfc, bfc, edge_index_0, edge_index_1, edge_index_2):
    raise NotImplementedError("write your pallas kernel here")


# speedup vs baseline: 3.5083x; 3.5083x over previous
import functools

import jax
import jax.numpy as jnp
from jax import lax
from jax.experimental import pallas as pl
from jax.experimental.pallas import tpu as pltpu
from jax.experimental.pallas import tpu_sc as plsc

_N = 10000
_D = 128
_E = 320000
_NT = 3
_NC = 2
_NS = 16
_NW = _NC * _NS
_EPW = _E // _NW
_B = 80
_NB = _EPW // _B
_RPT = 640
_RPT_LAST = _N - 15 * _RPT


def _sc_mesh():
    return plsc.VectorSubcoreMesh(core_axis_name="c", subcore_axis_name="s")


def _deg_sc(edges, zeros1):
    @functools.partial(
        pl.kernel,
        out_type=jax.ShapeDtypeStruct((_NC, 2 * _NT, _N), jnp.float32),
        mesh=_sc_mesh(),
        scratch_types=[pltpu.VMEM((_B,), jnp.int32),
                       pltpu.VMEM((_B,), jnp.float32)]
                      + [pltpu.VMEM_SHARED((_N,), jnp.float32)] * (2 * _NT),
    )
    def deg_kernel(e_hbm, z1_hbm, out_hbm, idx_v, ones_v, a0, a1, a2, a3, a4, a5):
        accs = [a0, a1, a2, a3, a4, a5]
        cid = lax.axis_index("c")
        sid = lax.axis_index("s")
        w = cid * _NS + sid
        for i in range(_B // 16):
            ones_v[pl.ds(16 * i, 16)] = jnp.ones((16,), jnp.float32)
        for k in range(2 * _NT):
            @pl.when(sid == k)
            def _():
                pltpu.sync_copy(z1_hbm, accs[k])
        plsc.subcore_barrier()
        base0 = w * _EPW
        for t in range(_NT):
            for side in range(2):
                acc = accs[2 * t + side]
                stream_base = (2 * t + side) * _E + base0

                def body(i, _):
                    b = stream_base + i * _B
                    pltpu.sync_copy(e_hbm.at[pl.ds(b, _B)], idx_v)
                    pltpu.sync_copy(ones_v, acc.at[idx_v], add=True)
                    return 0

                lax.fori_loop(0, _NB, body, 0)
        plsc.subcore_barrier()
        for k in range(2 * _NT):
            @pl.when(sid == k)
            def _():
                pltpu.sync_copy(accs[k], out_hbm.at[cid, k])

    return deg_kernel(edges, zeros1)


def _agg_sc(y0, y1, y2, edges, zeros2):
    @functools.partial(
        pl.kernel,
        out_type=jax.ShapeDtypeStruct((_NC, _NT, _N, _D), jnp.float32),
        mesh=_sc_mesh(),
        scratch_types=[pltpu.VMEM((_B,), jnp.int32),
                       pltpu.VMEM((_B,), jnp.int32),
                       pltpu.VMEM((_B, _D), jnp.float32),
                       pltpu.VMEM_SHARED((_N, _D), jnp.float32),
                       pltpu.SemaphoreType.DMA],
    )
    def agg_kernel(y0_hbm, y1_hbm, y2_hbm, e_hbm, z2_hbm, out_hbm,
                   src_v, dst_v, rows_v, acc_sh, sem):
        ys = [y0_hbm, y1_hbm, y2_hbm]
        cid = lax.axis_index("c")
        sid = lax.axis_index("s")
        w = cid * _NS + sid
        base0 = w * _EPW
        for t in range(_NT):
            @pl.when(sid < _NS - 1)
            def _():
                pltpu.sync_copy(z2_hbm.at[pl.ds(sid * _RPT, _RPT)],
                                acc_sh.at[pl.ds(sid * _RPT, _RPT)])

            @pl.when(sid == _NS - 1)
            def _():
                pltpu.sync_copy(z2_hbm.at[pl.ds(15 * _RPT, _RPT_LAST)],
                                acc_sh.at[pl.ds(15 * _RPT, _RPT_LAST)])

            plsc.subcore_barrier()
            y_hbm = ys[t]
            src_base = (2 * t + 0) * _E + base0
            dst_base = (2 * t + 1) * _E + base0

            def body(i, _):
                off = i * _B
                pltpu.sync_copy(e_hbm.at[pl.ds(src_base + off, _B)], src_v)
                pltpu.async_copy(y_hbm.at[src_v], rows_v, sem).wait()
                pltpu.sync_copy(e_hbm.at[pl.ds(dst_base + off, _B)], dst_v)
                pltpu.sync_copy(rows_v, acc_sh.at[dst_v], add=True)
                return 0

            lax.fori_loop(0, _NB, body, 0)
            plsc.subcore_barrier()

            @pl.when(sid < _NS - 1)
            def _():
                pltpu.sync_copy(acc_sh.at[pl.ds(sid * _RPT, _RPT)],
                                out_hbm.at[cid, t, pl.ds(sid * _RPT, _RPT)])

            @pl.when(sid == _NS - 1)
            def _():
                pltpu.sync_copy(acc_sh.at[pl.ds(15 * _RPT, _RPT_LAST)],
                                out_hbm.at[cid, t, pl.ds(15 * _RPT, _RPT_LAST)])

            plsc.subcore_barrier()

    return agg_kernel(y0, y1, y2, edges, zeros2)


_R = 2000


def _tc_layer0(x, dT, W0s):
    def body(x_ref, d_ref, w_ref, y_ref):
        xb = x_ref[...]
        for t in range(_NT):
            s = lax.rsqrt(jnp.maximum(d_ref[:, 2 * t:2 * t + 1], 1.0))
            y_ref[t] = jnp.dot(xb * s, w_ref[t],
                               preferred_element_type=jnp.float32)

    return pl.pallas_call(
        body,
        grid=(_N // _R,),
        in_specs=[pl.BlockSpec((_R, _D), lambda i: (i, 0)),
                  pl.BlockSpec((_R, 2 * _NT), lambda i: (i, 0)),
                  pl.BlockSpec((_NT, _D, _D), lambda i: (0, 0, 0))],
        out_specs=pl.BlockSpec((_NT, _R, _D), lambda i: (0, i, 0)),
        out_shape=jax.ShapeDtypeStruct((_NT, _N, _D), jnp.float32),
    )(x, dT, W0s)


def _tc_mid(z, dT, b0s, W1s):
    def body(z_ref, d_ref, b_ref, w_ref, y_ref):
        acc = jnp.zeros((_R, _D), jnp.float32)
        for t in range(_NT):
            zt = z_ref[0, t] + z_ref[1, t]
            sin = lax.rsqrt(jnp.maximum(d_ref[:, 2 * t + 1:2 * t + 2], 1.0))
            acc = acc + zt * sin
        h = acc + jnp.sum(b_ref[...], axis=0, keepdims=True)
        h = jnp.where(h >= 0, h, 0.01 * h)
        for t in range(_NT):
            sout = lax.rsqrt(jnp.maximum(d_ref[:, 2 * t:2 * t + 1], 1.0))
            y_ref[t] = jnp.dot(h * sout, w_ref[t],
                               preferred_element_type=jnp.float32)

    return pl.pallas_call(
        body,
        grid=(_N // _R,),
        in_specs=[pl.BlockSpec((_NC, _NT, _R, _D), lambda i: (0, 0, i, 0)),
                  pl.BlockSpec((_R, 2 * _NT), lambda i: (i, 0)),
                  pl.BlockSpec((_NT, _D), lambda i: (0, 0)),
                  pl.BlockSpec((_NT, _D, _D), lambda i: (0, 0, 0))],
        out_specs=pl.BlockSpec((_NT, _R, _D), lambda i: (0, i, 0)),
        out_shape=jax.ShapeDtypeStruct((_NT, _N, _D), jnp.float32),
    )(z, dT, b0s, W1s)


def _tc_final(z2, dT, b1s, Wfc_p, bfc_p):
    def body(z_ref, d_ref, b_ref, wfc_ref, bfc_ref, h2_ref, lg_ref):
        acc = jnp.zeros((_R, _D), jnp.float32)
        for t in range(_NT):
            zt = z_ref[0, t] + z_ref[1, t]
            sin = lax.rsqrt(jnp.maximum(d_ref[:, 2 * t + 1:2 * t + 2], 1.0))
            acc = acc + zt * sin
        h2 = acc + jnp.sum(b_ref[...], axis=0, keepdims=True)
        h2_ref[...] = h2
        lg_ref[...] = jnp.dot(h2, wfc_ref[...],
                              preferred_element_type=jnp.float32) + bfc_ref[...]

    return pl.pallas_call(
        body,
        grid=(_N // _R,),
        in_specs=[pl.BlockSpec((_NC, _NT, _R, _D), lambda i: (0, 0, i, 0)),
                  pl.BlockSpec((_R, 2 * _NT), lambda i: (i, 0)),
                  pl.BlockSpec((_NT, _D), lambda i: (0, 0)),
                  pl.BlockSpec((_D, _D), lambda i: (0, 0)),
                  pl.BlockSpec((1, _D), lambda i: (0, 0))],
        out_specs=[pl.BlockSpec((_R, _D), lambda i: (i, 0)),
                   pl.BlockSpec((_R, _D), lambda i: (i, 0))],
        out_shape=[jax.ShapeDtypeStruct((_N, _D), jnp.float32),
                   jax.ShapeDtypeStruct((_N, _D), jnp.float32)],
    )(z2, dT, b1s, Wfc_p, bfc_p)


def kernel(x, W0_0, b0_0, W0_1, b0_1, W0_2, b0_2,
           W1_0, b1_0, W1_1, b1_1, W1_2, b1_2, Wfc, bfc,
           edge_index_0, edge_index_1, edge_index_2):
    f32 = jnp.float32
    C = Wfc.shape[1]
    edges = jnp.stack([edge_index_0, edge_index_1,
                       edge_index_2]).reshape(-1)
    zeros1 = jnp.zeros((_N,), f32)
    zeros2 = jnp.zeros((_N, _D), f32)

    degs = _deg_sc(edges, zeros1)
    dT = (degs[0] + degs[1]).T

    W0s = jnp.stack([W0_0, W0_1, W0_2])
    b0s = jnp.stack([b0_0, b0_1, b0_2])
    W1s = jnp.stack([W1_0, W1_1, W1_2])
    b1s = jnp.stack([b1_0, b1_1, b1_2])

    y = _tc_layer0(x, dT, W0s)
    z = _agg_sc(y[0], y[1], y[2], edges, zeros2)
    y1 = _tc_mid(z, dT, b0s, W1s)
    z2 = _agg_sc(y1[0], y1[1], y1[2], edges, zeros2)

    Wfc_p = jnp.zeros((_D, _D), f32).at[:, :C].set(Wfc)
    bfc_p = jnp.zeros((1, _D), f32).at[0, :C].set(bfc)
    h2, logits_p = _tc_final(z2, dT, b1s, Wfc_p, bfc_p)
    return (h2, logits_p[:, :C])


# speedup vs baseline: 9.3098x; 2.6537x over previous
import functools

import jax
import jax.numpy as jnp
from jax import lax
from jax.experimental import pallas as pl
from jax.experimental.pallas import tpu as pltpu
from jax.experimental.pallas import tpu_sc as plsc

_N = 10000
_D = 128
_E = 320000
_NT = 3
_NC = 2
_NS = 16
_NW = _NC * _NS
_EPW = _E // _NW
_B = 80
_NB = _EPW // _B
_RPT = 640
_RPT_LAST = _N - 15 * _RPT


def _sc_mesh():
    return plsc.VectorSubcoreMesh(core_axis_name="c", subcore_axis_name="s")


_NBUF = 5
_NG = _NB // _NBUF
_AB = 40
_ANB = _EPW // _AB
_ANG = _ANB // _NBUF


def _deg_sc(edges, zeros1):
    @functools.partial(
        pl.kernel,
        out_type=jax.ShapeDtypeStruct((_NC, 2 * _NT, _N), jnp.float32),
        mesh=_sc_mesh(),
        scratch_types=[pltpu.VMEM((_NBUF, _B), jnp.int32),
                       pltpu.VMEM((_B,), jnp.float32)]
                      + [pltpu.VMEM_SHARED((_N,), jnp.float32)] * (2 * _NT)
                      + [pltpu.SemaphoreType.DMA] * (2 * _NBUF),
    )
    def deg_kernel(e_hbm, z1_hbm, out_hbm, idx_v, ones_v,
                   a0, a1, a2, a3, a4, a5, *sems):
        accs = [a0, a1, a2, a3, a4, a5]
        sem_i = sems[:_NBUF]
        sem_s = sems[_NBUF:]
        cid = lax.axis_index("c")
        sid = lax.axis_index("s")
        w = cid * _NS + sid
        for i in range(_B // 16):
            ones_v[pl.ds(16 * i, 16)] = jnp.ones((16,), jnp.float32)
        for k in range(2 * _NT):
            @pl.when(sid == k)
            def _():
                pltpu.sync_copy(z1_hbm, accs[k])
        plsc.subcore_barrier()
        base0 = w * _EPW
        for t in range(_NT):
            for side in range(2):
                acc = accs[2 * t + side]
                stream_base = (2 * t + side) * _E + base0

                def body(j, _, acc=acc, stream_base=stream_base):
                    descs = []
                    for b in range(_NBUF):
                        i = j * _NBUF + b

                        @pl.when(j > 0)
                        def _(b=b, acc=acc):
                            pltpu.make_async_copy(
                                ones_v, acc.at[idx_v.at[b]], sem_s[b]).wait()

                        descs.append(pltpu.async_copy(
                            e_hbm.at[pl.ds(stream_base + i * _B, _B)],
                            idx_v.at[b], sem_i[b]))
                    for b in range(_NBUF):
                        descs[b].wait()
                        pltpu.async_copy(ones_v, acc.at[idx_v.at[b]],
                                         sem_s[b], add=True)
                    return 0

                lax.fori_loop(0, _NG, body, 0)
                for b in range(_NBUF):
                    pltpu.make_async_copy(
                        ones_v, acc.at[idx_v.at[b]], sem_s[b]).wait()
        plsc.subcore_barrier()
        for k in range(2 * _NT):
            @pl.when(sid == k)
            def _():
                pltpu.sync_copy(accs[k], out_hbm.at[cid, k])

    return deg_kernel(edges, zeros1)


def _agg_sc(y0, y1, y2, edges, zeros2):
    @functools.partial(
        pl.kernel,
        out_type=jax.ShapeDtypeStruct((_NC, _NT, _N, _D), jnp.float32),
        mesh=_sc_mesh(),
        scratch_types=[pltpu.VMEM((_EPW,), jnp.int32),
                       pltpu.VMEM((_NBUF, _AB), jnp.int32),
                       pltpu.VMEM((_NBUF, _AB, _D), jnp.float32),
                       pltpu.VMEM_SHARED((_N, _D), jnp.float32)]
                      + [pltpu.SemaphoreType.DMA] * (3 * _NBUF),
    )
    def agg_kernel(y0_hbm, y1_hbm, y2_hbm, e_hbm, z2_hbm, out_hbm,
                   src_all, dst_v, rows_v, acc_sh, *sems):
        ys = [y0_hbm, y1_hbm, y2_hbm]
        sem_d = sems[:_NBUF]
        sem_g = sems[_NBUF:2 * _NBUF]
        sem_s = sems[2 * _NBUF:]
        cid = lax.axis_index("c")
        sid = lax.axis_index("s")
        w = cid * _NS + sid
        base0 = w * _EPW
        for t in range(_NT):
            @pl.when(sid < _NS - 1)
            def _():
                pltpu.sync_copy(z2_hbm.at[pl.ds(sid * _RPT, _RPT)],
                                acc_sh.at[pl.ds(sid * _RPT, _RPT)])

            @pl.when(sid == _NS - 1)
            def _():
                pltpu.sync_copy(z2_hbm.at[pl.ds(15 * _RPT, _RPT_LAST)],
                                acc_sh.at[pl.ds(15 * _RPT, _RPT_LAST)])

            plsc.subcore_barrier()
            y_hbm = ys[t]
            src_base = (2 * t + 0) * _E + base0
            dst_base = (2 * t + 1) * _E + base0
            pltpu.sync_copy(e_hbm.at[pl.ds(src_base, _EPW)], src_all)

            def body(j, _, y_hbm=y_hbm, dst_base=dst_base):
                gathers = []
                dloads = []
                for b in range(_NBUF):
                    i = j * _NBUF + b

                    @pl.when(j > 0)
                    def _(b=b):
                        pltpu.make_async_copy(
                            rows_v.at[b], acc_sh.at[dst_v.at[b]],
                            sem_s[b]).wait()

                    dloads.append(pltpu.async_copy(
                        e_hbm.at[pl.ds(dst_base + i * _AB, _AB)],
                        dst_v.at[b], sem_d[b]))
                    gathers.append(pltpu.async_copy(
                        y_hbm.at[src_all.at[pl.ds(i * _AB, _AB)]],
                        rows_v.at[b], sem_g[b]))
                for b in range(_NBUF):
                    gathers[b].wait()
                    dloads[b].wait()
                    pltpu.async_copy(rows_v.at[b], acc_sh.at[dst_v.at[b]],
                                     sem_s[b], add=True)
                return 0

            lax.fori_loop(0, _ANG, body, 0)
            for b in range(_NBUF):
                pltpu.make_async_copy(rows_v.at[b], acc_sh.at[dst_v.at[b]],
                                      sem_s[b]).wait()
            plsc.subcore_barrier()

            @pl.when(sid < _NS - 1)
            def _():
                pltpu.sync_copy(acc_sh.at[pl.ds(sid * _RPT, _RPT)],
                                out_hbm.at[cid, t, pl.ds(sid * _RPT, _RPT)])

            @pl.when(sid == _NS - 1)
            def _():
                pltpu.sync_copy(acc_sh.at[pl.ds(15 * _RPT, _RPT_LAST)],
                                out_hbm.at[cid, t, pl.ds(15 * _RPT, _RPT_LAST)])

            plsc.subcore_barrier()

    return agg_kernel(y0, y1, y2, edges, zeros2)


_R = 2000


def _tc_layer0(x, dT, W0s):
    def body(x_ref, d_ref, w_ref, y_ref):
        xb = x_ref[...]
        for t in range(_NT):
            s = lax.rsqrt(jnp.maximum(d_ref[:, 2 * t:2 * t + 1], 1.0))
            y_ref[t] = jnp.dot(xb * s, w_ref[t],
                               preferred_element_type=jnp.float32)

    return pl.pallas_call(
        body,
        grid=(_N // _R,),
        in_specs=[pl.BlockSpec((_R, _D), lambda i: (i, 0)),
                  pl.BlockSpec((_R, 2 * _NT), lambda i: (i, 0)),
                  pl.BlockSpec((_NT, _D, _D), lambda i: (0, 0, 0))],
        out_specs=pl.BlockSpec((_NT, _R, _D), lambda i: (0, i, 0)),
        out_shape=jax.ShapeDtypeStruct((_NT, _N, _D), jnp.float32),
    )(x, dT, W0s)


def _tc_mid(z, dT, b0s, W1s):
    def body(z_ref, d_ref, b_ref, w_ref, y_ref):
        acc = jnp.zeros((_R, _D), jnp.float32)
        for t in range(_NT):
            zt = z_ref[0, t] + z_ref[1, t]
            sin = lax.rsqrt(jnp.maximum(d_ref[:, 2 * t + 1:2 * t + 2], 1.0))
            acc = acc + zt * sin
        h = acc + jnp.sum(b_ref[...], axis=0, keepdims=True)
        h = jnp.where(h >= 0, h, 0.01 * h)
        for t in range(_NT):
            sout = lax.rsqrt(jnp.maximum(d_ref[:, 2 * t:2 * t + 1], 1.0))
            y_ref[t] = jnp.dot(h * sout, w_ref[t],
                               preferred_element_type=jnp.float32)

    return pl.pallas_call(
        body,
        grid=(_N // _R,),
        in_specs=[pl.BlockSpec((_NC, _NT, _R, _D), lambda i: (0, 0, i, 0)),
                  pl.BlockSpec((_R, 2 * _NT), lambda i: (i, 0)),
                  pl.BlockSpec((_NT, _D), lambda i: (0, 0)),
                  pl.BlockSpec((_NT, _D, _D), lambda i: (0, 0, 0))],
        out_specs=pl.BlockSpec((_NT, _R, _D), lambda i: (0, i, 0)),
        out_shape=jax.ShapeDtypeStruct((_NT, _N, _D), jnp.float32),
    )(z, dT, b0s, W1s)


def _tc_final(z2, dT, b1s, Wfc_p, bfc_p):
    def body(z_ref, d_ref, b_ref, wfc_ref, bfc_ref, h2_ref, lg_ref):
        acc = jnp.zeros((_R, _D), jnp.float32)
        for t in range(_NT):
            zt = z_ref[0, t] + z_ref[1, t]
            sin = lax.rsqrt(jnp.maximum(d_ref[:, 2 * t + 1:2 * t + 2], 1.0))
            acc = acc + zt * sin
        h2 = acc + jnp.sum(b_ref[...], axis=0, keepdims=True)
        h2_ref[...] = h2
        lg_ref[...] = jnp.dot(h2, wfc_ref[...],
                              preferred_element_type=jnp.float32) + bfc_ref[...]

    return pl.pallas_call(
        body,
        grid=(_N // _R,),
        in_specs=[pl.BlockSpec((_NC, _NT, _R, _D), lambda i: (0, 0, i, 0)),
                  pl.BlockSpec((_R, 2 * _NT), lambda i: (i, 0)),
                  pl.BlockSpec((_NT, _D), lambda i: (0, 0)),
                  pl.BlockSpec((_D, _D), lambda i: (0, 0)),
                  pl.BlockSpec((1, _D), lambda i: (0, 0))],
        out_specs=[pl.BlockSpec((_R, _D), lambda i: (i, 0)),
                   pl.BlockSpec((_R, _D), lambda i: (i, 0))],
        out_shape=[jax.ShapeDtypeStruct((_N, _D), jnp.float32),
                   jax.ShapeDtypeStruct((_N, _D), jnp.float32)],
    )(z2, dT, b1s, Wfc_p, bfc_p)


def kernel(x, W0_0, b0_0, W0_1, b0_1, W0_2, b0_2,
           W1_0, b1_0, W1_1, b1_1, W1_2, b1_2, Wfc, bfc,
           edge_index_0, edge_index_1, edge_index_2):
    f32 = jnp.float32
    C = Wfc.shape[1]
    edges = jnp.stack([edge_index_0, edge_index_1,
                       edge_index_2]).reshape(-1)
    zeros1 = jnp.zeros((_N,), f32)
    zeros2 = jnp.zeros((_N, _D), f32)

    degs = _deg_sc(edges, zeros1)
    dT = (degs[0] + degs[1]).T

    W0s = jnp.stack([W0_0, W0_1, W0_2])
    b0s = jnp.stack([b0_0, b0_1, b0_2])
    W1s = jnp.stack([W1_0, W1_1, W1_2])
    b1s = jnp.stack([b1_0, b1_1, b1_2])

    y = _tc_layer0(x, dT, W0s)
    z = _agg_sc(y[0], y[1], y[2], edges, zeros2)
    y1 = _tc_mid(z, dT, b0s, W1s)
    z2 = _agg_sc(y1[0], y1[1], y1[2], edges, zeros2)

    Wfc_p = jnp.zeros((_D, _D), f32).at[:, :C].set(Wfc)
    bfc_p = jnp.zeros((1, _D), f32).at[0, :C].set(bfc)
    h2, logits_p = _tc_final(z2, dT, b1s, Wfc_p, bfc_p)
    return (h2, logits_p[:, :C])


# speedup vs baseline: 9.6832x; 1.0401x over previous
import functools

import jax
import jax.numpy as jnp
from jax import lax
from jax.experimental import pallas as pl
from jax.experimental.pallas import tpu as pltpu
from jax.experimental.pallas import tpu_sc as plsc

_N = 10000
_D = 128
_E = 320000
_NT = 3
_NC = 2
_NS = 16
_NW = _NC * _NS
_EPW = _E // _NW
_B = 80
_NB = _EPW // _B
_RPT = 640
_RPT_LAST = _N - 15 * _RPT


def _sc_mesh():
    return plsc.VectorSubcoreMesh(core_axis_name="c", subcore_axis_name="s")


_NBUF = 5
_NG = _NB // _NBUF
_AB = 40
_ANB = _EPW // _AB
_ANG = _ANB // _NBUF
_ZR = 80


def _deg_sc(edges, zeros1):
    @functools.partial(
        pl.kernel,
        out_type=jax.ShapeDtypeStruct((_NC, 2 * _NT, _N), jnp.float32),
        mesh=_sc_mesh(),
        scratch_types=[pltpu.VMEM((_NBUF, _B), jnp.int32),
                       pltpu.VMEM((_B,), jnp.float32)]
                      + [pltpu.VMEM_SHARED((_N,), jnp.float32)] * (2 * _NT)
                      + [pltpu.SemaphoreType.DMA] * (2 * _NBUF),
    )
    def deg_kernel(e0_hbm, e1_hbm, e2_hbm, z1_hbm, out_hbm, idx_v, ones_v,
                   a0, a1, a2, a3, a4, a5, *sems):
        es = [e0_hbm, e1_hbm, e2_hbm]
        accs = [a0, a1, a2, a3, a4, a5]
        sem_i = sems[:_NBUF]
        sem_s = sems[_NBUF:]
        cid = lax.axis_index("c")
        sid = lax.axis_index("s")
        w = cid * _NS + sid
        for i in range(_B // 16):
            ones_v[pl.ds(16 * i, 16)] = jnp.ones((16,), jnp.float32)
        for k in range(2 * _NT):
            @pl.when(sid == k)
            def _():
                pltpu.sync_copy(z1_hbm, accs[k])
        plsc.subcore_barrier()
        base0 = w * _EPW
        for t in range(_NT):
            for side in range(2):
                acc = accs[2 * t + side]
                e_hbm = es[t]
                stream_base = side * _E + base0

                def body(j, _, acc=acc, stream_base=stream_base, e_hbm=e_hbm):
                    descs = []
                    for b in range(_NBUF):
                        i = j * _NBUF + b

                        @pl.when(j > 0)
                        def _(b=b, acc=acc):
                            pltpu.make_async_copy(
                                ones_v, acc.at[idx_v.at[b]], sem_s[b]).wait()

                        descs.append(pltpu.async_copy(
                            e_hbm.at[pl.ds(stream_base + i * _B, _B)],
                            idx_v.at[b], sem_i[b]))
                    for b in range(_NBUF):
                        descs[b].wait()
                        pltpu.async_copy(ones_v, acc.at[idx_v.at[b]],
                                         sem_s[b], add=True)
                    return 0

                lax.fori_loop(0, _NG, body, 0)
                for b in range(_NBUF):
                    pltpu.make_async_copy(
                        ones_v, acc.at[idx_v.at[b]], sem_s[b]).wait()
        plsc.subcore_barrier()
        for k in range(2 * _NT):
            @pl.when(sid == k)
            def _():
                pltpu.sync_copy(accs[k], out_hbm.at[cid, k])

    return deg_kernel(edges[0], edges[1], edges[2], zeros1)


def _agg_sc(y0, y1, y2, edges):
    @functools.partial(
        pl.kernel,
        out_type=jax.ShapeDtypeStruct((_NC, _NT, _N, _D), jnp.float32),
        mesh=_sc_mesh(),
        scratch_types=[pltpu.VMEM((_EPW,), jnp.int32),
                       pltpu.VMEM((_NBUF, _AB), jnp.int32),
                       pltpu.VMEM((_NBUF, _AB, _D), jnp.float32),
                       pltpu.VMEM((_ZR, _D), jnp.float32),
                       pltpu.VMEM_SHARED((_N, _D), jnp.float32)]
                      + [pltpu.SemaphoreType.DMA] * (3 * _NBUF),
    )
    def agg_kernel(y0_hbm, y1_hbm, y2_hbm, e0_hbm, e1_hbm, e2_hbm, out_hbm,
                   src_all, dst_v, rows_v, zbuf_v, acc_sh, *sems):
        ys = [y0_hbm, y1_hbm, y2_hbm]
        es = [e0_hbm, e1_hbm, e2_hbm]
        sem_d = sems[:_NBUF]
        sem_g = sems[_NBUF:2 * _NBUF]
        sem_s = sems[2 * _NBUF:]
        cid = lax.axis_index("c")
        sid = lax.axis_index("s")
        w = cid * _NS + sid
        base0 = w * _EPW

        def zb_body(i, _):
            for k in range(_D // 16):
                zbuf_v[i, pl.ds(16 * k, 16)] = jnp.zeros((16,), jnp.float32)
            return 0

        lax.fori_loop(0, _ZR, zb_body, 0)
        for t in range(_NT):
            nz = _RPT // _ZR

            def z_body(i, _):
                pltpu.sync_copy(
                    zbuf_v, acc_sh.at[pl.ds(sid * _RPT + i * _ZR, _ZR)])
                return 0

            @pl.when(sid < _NS - 1)
            def _():
                lax.fori_loop(0, nz, z_body, 0)

            @pl.when(sid == _NS - 1)
            def _():
                lax.fori_loop(0, _RPT_LAST // _ZR, z_body, 0)

            plsc.subcore_barrier()
            y_hbm = ys[t]
            e_hbm = es[t]
            src_base = base0
            dst_base = _E + base0
            pltpu.sync_copy(e_hbm.at[pl.ds(src_base, _EPW)], src_all)

            def body(j, _, y_hbm=y_hbm, dst_base=dst_base, e_hbm=e_hbm):
                gathers = []
                dloads = []
                for b in range(_NBUF):
                    i = j * _NBUF + b

                    @pl.when(j > 0)
                    def _(b=b):
                        pltpu.make_async_copy(
                            rows_v.at[b], acc_sh.at[dst_v.at[b]],
                            sem_s[b]).wait()

                    dloads.append(pltpu.async_copy(
                        e_hbm.at[pl.ds(dst_base + i * _AB, _AB)],
                        dst_v.at[b], sem_d[b]))
                    gathers.append(pltpu.async_copy(
                        y_hbm.at[src_all.at[pl.ds(i * _AB, _AB)]],
                        rows_v.at[b], sem_g[b]))
                for b in range(_NBUF):
                    gathers[b].wait()
                    dloads[b].wait()
                    pltpu.async_copy(rows_v.at[b], acc_sh.at[dst_v.at[b]],
                                     sem_s[b], add=True)
                return 0

            lax.fori_loop(0, _ANG, body, 0)
            for b in range(_NBUF):
                pltpu.make_async_copy(rows_v.at[b], acc_sh.at[dst_v.at[b]],
                                      sem_s[b]).wait()
            plsc.subcore_barrier()

            @pl.when(sid < _NS - 1)
            def _():
                pltpu.sync_copy(acc_sh.at[pl.ds(sid * _RPT, _RPT)],
                                out_hbm.at[cid, t, pl.ds(sid * _RPT, _RPT)])

            @pl.when(sid == _NS - 1)
            def _():
                pltpu.sync_copy(acc_sh.at[pl.ds(15 * _RPT, _RPT_LAST)],
                                out_hbm.at[cid, t, pl.ds(15 * _RPT, _RPT_LAST)])

            plsc.subcore_barrier()

    return agg_kernel(y0, y1, y2, edges[0], edges[1], edges[2])


_R = 2000


def _tc_layer0(x, dT, W0s):
    def body(x_ref, d_ref, w_ref, y_ref):
        xb = x_ref[...]
        for t in range(_NT):
            s = lax.rsqrt(jnp.maximum(d_ref[:, 2 * t:2 * t + 1], 1.0))
            y_ref[t] = jnp.dot(xb * s, w_ref[t],
                               preferred_element_type=jnp.float32)

    return pl.pallas_call(
        body,
        grid=(_N // _R,),
        in_specs=[pl.BlockSpec((_R, _D), lambda i: (i, 0)),
                  pl.BlockSpec((_R, 2 * _NT), lambda i: (i, 0)),
                  pl.BlockSpec((_NT, _D, _D), lambda i: (0, 0, 0))],
        out_specs=pl.BlockSpec((_NT, _R, _D), lambda i: (0, i, 0)),
        out_shape=jax.ShapeDtypeStruct((_NT, _N, _D), jnp.float32),
    )(x, dT, W0s)


def _tc_mid(z, dT, b0s, W1s):
    def body(z_ref, d_ref, b_ref, w_ref, y_ref):
        acc = jnp.zeros((_R, _D), jnp.float32)
        for t in range(_NT):
            zt = z_ref[0, t] + z_ref[1, t]
            sin = lax.rsqrt(jnp.maximum(d_ref[:, 2 * t + 1:2 * t + 2], 1.0))
            acc = acc + zt * sin
        h = acc + jnp.sum(b_ref[...], axis=0, keepdims=True)
        h = jnp.where(h >= 0, h, 0.01 * h)
        for t in range(_NT):
            sout = lax.rsqrt(jnp.maximum(d_ref[:, 2 * t:2 * t + 1], 1.0))
            y_ref[t] = jnp.dot(h * sout, w_ref[t],
                               preferred_element_type=jnp.float32)

    return pl.pallas_call(
        body,
        grid=(_N // _R,),
        in_specs=[pl.BlockSpec((_NC, _NT, _R, _D), lambda i: (0, 0, i, 0)),
                  pl.BlockSpec((_R, 2 * _NT), lambda i: (i, 0)),
                  pl.BlockSpec((_NT, _D), lambda i: (0, 0)),
                  pl.BlockSpec((_NT, _D, _D), lambda i: (0, 0, 0))],
        out_specs=pl.BlockSpec((_NT, _R, _D), lambda i: (0, i, 0)),
        out_shape=jax.ShapeDtypeStruct((_NT, _N, _D), jnp.float32),
    )(z, dT, b0s, W1s)


def _tc_final(z2, dT, b1s, Wfc_p, bfc_p):
    def body(z_ref, d_ref, b_ref, wfc_ref, bfc_ref, h2_ref, lg_ref):
        acc = jnp.zeros((_R, _D), jnp.float32)
        for t in range(_NT):
            zt = z_ref[0, t] + z_ref[1, t]
            sin = lax.rsqrt(jnp.maximum(d_ref[:, 2 * t + 1:2 * t + 2], 1.0))
            acc = acc + zt * sin
        h2 = acc + jnp.sum(b_ref[...], axis=0, keepdims=True)
        h2_ref[...] = h2
        lg_ref[...] = jnp.dot(h2, wfc_ref[...],
                              preferred_element_type=jnp.float32) + bfc_ref[...]

    return pl.pallas_call(
        body,
        grid=(_N // _R,),
        in_specs=[pl.BlockSpec((_NC, _NT, _R, _D), lambda i: (0, 0, i, 0)),
                  pl.BlockSpec((_R, 2 * _NT), lambda i: (i, 0)),
                  pl.BlockSpec((_NT, _D), lambda i: (0, 0)),
                  pl.BlockSpec((_D, _D), lambda i: (0, 0)),
                  pl.BlockSpec((1, _D), lambda i: (0, 0))],
        out_specs=[pl.BlockSpec((_R, _D), lambda i: (i, 0)),
                   pl.BlockSpec((_R, _D), lambda i: (i, 0))],
        out_shape=[jax.ShapeDtypeStruct((_N, _D), jnp.float32),
                   jax.ShapeDtypeStruct((_N, _D), jnp.float32)],
    )(z2, dT, b1s, Wfc_p, bfc_p)


def kernel(x, W0_0, b0_0, W0_1, b0_1, W0_2, b0_2,
           W1_0, b1_0, W1_1, b1_1, W1_2, b1_2, Wfc, bfc,
           edge_index_0, edge_index_1, edge_index_2):
    f32 = jnp.float32
    C = Wfc.shape[1]
    edges = [edge_index_0.reshape(-1), edge_index_1.reshape(-1),
             edge_index_2.reshape(-1)]
    zeros1 = jnp.zeros((_N,), f32)

    degs = _deg_sc(edges, zeros1)
    dT = (degs[0] + degs[1]).T

    W0s = jnp.stack([W0_0, W0_1, W0_2])
    b0s = jnp.stack([b0_0, b0_1, b0_2])
    W1s = jnp.stack([W1_0, W1_1, W1_2])
    b1s = jnp.stack([b1_0, b1_1, b1_2])

    y = _tc_layer0(x, dT, W0s)
    z = _agg_sc(y[0], y[1], y[2], edges)
    y1 = _tc_mid(z, dT, b0s, W1s)
    z2 = _agg_sc(y1[0], y1[1], y1[2], edges)

    Wfc_p = jnp.zeros((_D, _D), f32).at[:, :C].set(Wfc)
    bfc_p = jnp.zeros((1, _D), f32).at[0, :C].set(bfc)
    h2, logits_p = _tc_final(z2, dT, b1s, Wfc_p, bfc_p)
    return (h2, logits_p[:, :C])
